# baseline (speedup 1.0000x reference)
import dataclasses
import functools

import jax
import jax.numpy as jnp
from jax import lax
from jax.experimental import pallas as pl
from jax.experimental.pallas import tpu as pltpu
from jax.experimental.pallas import tpu_sc as plsc

N = 10000
E = 160000
F = 256
FH = 128
NP = 10240
NR = 10240
ALPHA = 0.2
EPS = 1e-5
NC = 2
NS = 16
CH = 80
NCHUNK = E // CH
RPT = NR // NS

_sc_mesh = plsc.VectorSubcoreMesh(core_axis_name="c", subcore_axis_name="s")

_sc_cp = pltpu.CompilerParams()
if "needs_layout_passes" in pltpu.CompilerParams.__dataclass_fields__:
    _sc_cp = dataclasses.replace(_sc_cp, needs_layout_passes=False)


EPT = E // (NC * NS)
EPT_FULL = (EPT // 16) * 16


@functools.partial(
    pl.kernel,
    out_type=jax.ShapeDtypeStruct((NC * NS * NP,), jnp.float32),
    mesh=_sc_mesh,
    compiler_params=_sc_cp,
    scratch_types=[
        pltpu.VMEM((NP,), jnp.float32),
        pltpu.VMEM((EPT_FULL + 16,), jnp.int32),
    ],
)
def _sc_degree(ef_hbm, degp_hbm, hist_v, idx_v):
    cc = lax.axis_index("c")
    sid = lax.axis_index("s")
    ones = jnp.full((16,), 1.0, jnp.float32)

    @pl.loop(0, NP, step=16)
    def _zero(i):
        hist_v[pl.ds(i, 16)] = jnp.zeros((16,), jnp.float32)

    base_e = E + (cc * NS + sid) * EPT
    pltpu.sync_copy(ef_hbm.at[pl.ds(base_e, EPT)], idx_v.at[pl.ds(0, EPT)])

    @pl.loop(0, EPT_FULL, step=64)
    def _scat(i):
        for u in range(4):
            idx = idx_v[pl.ds(i + u * 16, 16)]
            plsc.addupdate_scatter(hist_v, [idx], ones)

    tail = EPT - EPT_FULL
    if tail:
        mask = jnp.arange(16, dtype=jnp.int32) < tail
        idx = jnp.where(mask, idx_v[pl.ds(EPT_FULL, 16)], 0)
        plsc.addupdate_scatter(hist_v, [idx], ones, mask=mask)

    pltpu.sync_copy(hist_v, degp_hbm.at[pl.ds((cc * NS + sid) * NP, NP)])


NBUF = 4
NGRP = 31
TPT = NGRP * NBUF


@functools.partial(
    pl.kernel,
    out_type=[
        jax.ShapeDtypeStruct((NR, FH), jnp.float32),
        jax.ShapeDtypeStruct((NR, FH), jnp.float32),
    ],
    mesh=_sc_mesh,
    compiler_params=_sc_cp,
    scratch_types=[
        pltpu.VMEM((NBUF, 2, 2, CH), jnp.int32),
        pltpu.VMEM((NBUF, CH, FH), jnp.float32),
        pltpu.VMEM_SHARED((NR, FH), jnp.float32),
        pltpu.SemaphoreType.DMA((NBUF, 2)),
        pltpu.SemaphoreType.DMA((NBUF,)),
        pltpu.SemaphoreType.DMA((NBUF,)),
    ],
)
def _sc_aggregate(ef_hbm, xs_hbm, agg0_hbm, agg1_hbm, idx_v, rows_v, agg_sh,
                  sem_i, sem_g, sem_s):
    cc = lax.axis_index("c")
    sid = lax.axis_index("s")
    row0 = sid * RPT
    sbase = cc * E

    def _fetch_idx(j, b, p, sem):
        pltpu.async_copy(ef_hbm.at[pl.ds(sbase + j * CH, CH)], idx_v.at[b, p, 0], sem)
        pltpu.async_copy(ef_hbm.at[pl.ds(2 * E + j * CH, CH)], idx_v.at[b, p, 1], sem)

    def _wait_idx(b, p):
        pltpu.make_async_copy(ef_hbm.at[pl.ds(0, CH)], idx_v.at[b, p, 0],
                              sem_i.at[b, p]).wait()
        pltpu.make_async_copy(ef_hbm.at[pl.ds(0, CH)], idx_v.at[b, p, 1],
                              sem_i.at[b, p]).wait()

    pltpu.sync_copy(
        xs_hbm.at[pl.ds(cc * NR + row0, RPT)], agg_sh.at[pl.ds(row0, RPT)]
    )
    plsc.subcore_barrier()

    def _wait_scatter(b, p):
        pltpu.make_async_copy(
            rows_v.at[b], agg_sh.at[idx_v.at[b, p, 1]], sem_s.at[b]
        ).wait()

    for b in range(NBUF):
        _fetch_idx(b * NS + sid, b, 0, sem_i.at[b, 0])

    def _run_group(g, phase, first=False):
        for b in range(NBUF):
            if not first:
                _wait_scatter(b, phase)
            _wait_idx(b, phase)
            pltpu.async_copy(
                xs_hbm.at[idx_v.at[b, phase, 0]], rows_v.at[b], sem_g.at[b]
            )
            jn = ((g + 1) * NBUF + b) * NS + sid

            @pl.when(jn < TPT * NS)
            def _():
                _fetch_idx(jn, b, 1 - phase, sem_i.at[b, 1 - phase])
        for b in range(NBUF):
            pltpu.make_async_copy(
                xs_hbm.at[idx_v.at[b, phase, 0]], rows_v.at[b], sem_g.at[b]
            ).wait()
            pltpu.async_copy(
                rows_v.at[b], agg_sh.at[idx_v.at[b, phase, 1]], sem_s.at[b],
                add=True,
            )

    @pl.loop(0, (NGRP - 1) // 2)
    def _group2(gg):
        for b in range(NBUF):
            @pl.when(gg > 0)
            def _():
                _wait_scatter(b, 0)
            _wait_idx(b, 0)
            pltpu.async_copy(
                xs_hbm.at[idx_v.at[b, 0, 0]], rows_v.at[b], sem_g.at[b]
            )
            jn = (gg * 2 * NBUF + NBUF + b) * NS + sid
            _fetch_idx(jn, b, 1, sem_i.at[b, 1])
        for b in range(NBUF):
            pltpu.make_async_copy(
                xs_hbm.at[idx_v.at[b, 0, 0]], rows_v.at[b], sem_g.at[b]
            ).wait()
            pltpu.async_copy(
                rows_v.at[b], agg_sh.at[idx_v.at[b, 0, 1]], sem_s.at[b],
                add=True,
            )
        _run_group(gg * 2 + 1, 1)

    _run_group(NGRP - 1, 0)

    for b in range(NBUF):
        _wait_scatter(b, 0)

    for r in range((NCHUNK - TPT * NS) // NS):
        j = (TPT + r) * NS + sid
        pltpu.sync_copy(ef_hbm.at[pl.ds(sbase + j * CH, CH)], idx_v.at[0, 0, 0])
        pltpu.sync_copy(ef_hbm.at[pl.ds(2 * E + j * CH, CH)], idx_v.at[0, 0, 1])
        pltpu.sync_copy(xs_hbm.at[idx_v.at[0, 0, 0]], rows_v.at[0])
        pltpu.sync_copy(rows_v.at[0], agg_sh.at[idx_v.at[0, 0, 1]], add=True)

    plsc.subcore_barrier()

    @pl.when(cc == 0)
    def _out0():
        pltpu.sync_copy(agg_sh.at[pl.ds(row0, RPT)], agg0_hbm.at[pl.ds(row0, RPT)])

    @pl.when(cc == 1)
    def _out1():
        pltpu.sync_copy(agg_sh.at[pl.ds(row0, RPT)], agg1_hbm.at[pl.ds(row0, RPT)])


def _tc_prep_body(x_ref, degp_ref, xs_ref, dinv_ref):
    deg = jnp.sum(degp_ref[...], axis=0) + 1.0
    dinv = lax.rsqrt(deg)
    dcol = dinv[:N].reshape(N, 1)
    dinv_ref[...] = dcol
    xs_ref[0:N, :] = x_ref[:, 0:FH] * dcol
    xs_ref[N:NR, :] = jnp.zeros((NR - N, FH), jnp.float32)
    xs_ref[NR : NR + N, :] = x_ref[:, FH:F] * dcol
    xs_ref[NR + N :, :] = jnp.zeros((NR - N, FH), jnp.float32)


_tc_prep = pl.pallas_call(
    _tc_prep_body,
    out_shape=[
        jax.ShapeDtypeStruct((NC * NR, FH), jnp.float32),
        jax.ShapeDtypeStruct((N, 1), jnp.float32),
    ],
)


BF = 1000
NBF = N // BF


def _tc_final_body(agg0_ref, agg1_ref, dinv_ref, w_ref, b_ref, g_ref, be_ref,
                   out_ref, t_ref, acc_ref):
    p = pl.program_id(0)
    i = pl.program_id(1)

    @pl.when(p == 0)
    def _stats():
        @pl.when(i == 0)
        def _init():
            acc_ref[...] = jnp.zeros((2, F), jnp.float32)

        dcol = dinv_ref[...]
        z0 = (agg0_ref[...] * dcol).astype(jnp.bfloat16)
        z1 = (agg1_ref[...] * dcol).astype(jnp.bfloat16)
        t = (
            jnp.dot(z0, w_ref[0:FH, :].astype(jnp.bfloat16),
                    preferred_element_type=jnp.float32)
            + jnp.dot(z1, w_ref[FH:F, :].astype(jnp.bfloat16),
                      preferred_element_type=jnp.float32)
            + b_ref[...]
        )
        t_ref[pl.ds(i * BF, BF), :] = t
        acc_ref[0:1, :] = acc_ref[0:1, :] + jnp.sum(t, axis=0, keepdims=True)
        acc_ref[1:2, :] = acc_ref[1:2, :] + jnp.sum(t * t, axis=0, keepdims=True)

    @pl.when(p == 1)
    def _norm():
        mean = acc_ref[0:1, :] * (1.0 / N)
        var = acc_ref[1:2, :] * (1.0 / N) - mean * mean
        t = t_ref[pl.ds(i * BF, BF), :]
        o = (t - mean) * lax.rsqrt(var + EPS) * g_ref[...] + be_ref[...]
        out_ref[...] = jnp.where(o >= 0, o, ALPHA * o)


_tc_final = pl.pallas_call(
    _tc_final_body,
    grid=(2, NBF),
    in_specs=[
        pl.BlockSpec((BF, FH), lambda p, i: (i, 0)),
        pl.BlockSpec((BF, FH), lambda p, i: (i, 0)),
        pl.BlockSpec((BF, 1), lambda p, i: (i, 0)),
        pl.BlockSpec((F, F), lambda p, i: (0, 0)),
        pl.BlockSpec((F,), lambda p, i: (0,)),
        pl.BlockSpec((F,), lambda p, i: (0,)),
        pl.BlockSpec((F,), lambda p, i: (0,)),
    ],
    out_specs=pl.BlockSpec((BF, F), lambda p, i: (i * p, 0)),
    out_shape=jax.ShapeDtypeStruct((N, F), jnp.float32),
    scratch_shapes=[
        pltpu.VMEM((N, F), jnp.float32),
        pltpu.VMEM((2, F), jnp.float32),
    ],
)


def kernel(x, edge_idx, W, b, gamma, beta):
    ef2 = edge_idx.reshape(2 * E)
    ef3 = jnp.concatenate([edge_idx[0], edge_idx[0] + NR, edge_idx[1]])
    degp = _sc_degree(ef2).reshape(NC * NS, NP)
    xs, dinv = _tc_prep(x, degp)
    agg0, agg1 = _sc_aggregate(ef3, xs)
    return _tc_final(agg0, agg1, dinv, W, b, gamma, beta)


# speedup vs baseline: 1.0357x; 1.0357x over previous
import dataclasses
import functools

import jax
import jax.numpy as jnp
from jax import lax
from jax.experimental import pallas as pl
from jax.experimental.pallas import tpu as pltpu
from jax.experimental.pallas import tpu_sc as plsc

N = 10000
E = 160000
F = 256
FH = 128
NP = 10240
NR = 10240
ALPHA = 0.2
EPS = 1e-5
NC = 2
NS = 16
CH = 80
NCHUNK = E // CH
RPT = NR // NS

_sc_mesh = plsc.VectorSubcoreMesh(core_axis_name="c", subcore_axis_name="s")

_sc_cp = pltpu.CompilerParams()
if "needs_layout_passes" in pltpu.CompilerParams.__dataclass_fields__:
    _sc_cp = dataclasses.replace(_sc_cp, needs_layout_passes=False)


EPT = E // (NC * NS)
EPT_FULL = (EPT // 16) * 16


@functools.partial(
    pl.kernel,
    out_type=jax.ShapeDtypeStruct((NC * NS * NP,), jnp.float32),
    mesh=_sc_mesh,
    compiler_params=_sc_cp,
    scratch_types=[
        pltpu.VMEM((NP,), jnp.float32),
        pltpu.VMEM((EPT_FULL + 16,), jnp.int32),
    ],
)
def _sc_degree(ef_hbm, degp_hbm, hist_v, idx_v):
    cc = lax.axis_index("c")
    sid = lax.axis_index("s")
    ones = jnp.full((16,), 1.0, jnp.float32)

    @pl.loop(0, NP, step=128)
    def _zero(i):
        for u in range(8):
            hist_v[pl.ds(i + u * 16, 16)] = jnp.zeros((16,), jnp.float32)

    base_e = E + (cc * NS + sid) * EPT
    pltpu.sync_copy(ef_hbm.at[pl.ds(base_e, EPT)], idx_v.at[pl.ds(0, EPT)])

    @pl.loop(0, EPT_FULL, step=64)
    def _scat(i):
        for u in range(4):
            idx = idx_v[pl.ds(i + u * 16, 16)]
            plsc.addupdate_scatter(hist_v, [idx], ones)

    tail = EPT - EPT_FULL
    if tail:
        mask = jnp.arange(16, dtype=jnp.int32) < tail
        idx = jnp.where(mask, idx_v[pl.ds(EPT_FULL, 16)], 0)
        plsc.addupdate_scatter(hist_v, [idx], ones, mask=mask)

    pltpu.sync_copy(hist_v, degp_hbm.at[pl.ds((cc * NS + sid) * NP, NP)])


NBUF = 4
NGRP = 31
TPT = NGRP * NBUF


@functools.partial(
    pl.kernel,
    out_type=[
        jax.ShapeDtypeStruct((NR, FH), jnp.float32),
        jax.ShapeDtypeStruct((NR, FH), jnp.float32),
    ],
    mesh=_sc_mesh,
    compiler_params=_sc_cp,
    scratch_types=[
        pltpu.VMEM((NBUF, 2, 2, CH), jnp.int32),
        pltpu.VMEM((NBUF, CH, FH), jnp.float32),
        pltpu.VMEM_SHARED((NR, FH), jnp.float32),
        pltpu.SemaphoreType.DMA((NBUF, 2)),
        pltpu.SemaphoreType.DMA((NBUF,)),
        pltpu.SemaphoreType.DMA((NBUF,)),
    ],
)
def _sc_aggregate(ef_hbm, xs_hbm, agg0_hbm, agg1_hbm, idx_v, rows_v, agg_sh,
                  sem_i, sem_g, sem_s):
    cc = lax.axis_index("c")
    sid = lax.axis_index("s")
    row0 = sid * RPT
    sbase = cc * E

    def _fetch_idx(j, b, p, sem):
        pltpu.async_copy(ef_hbm.at[pl.ds(sbase + j * CH, CH)], idx_v.at[b, p, 0], sem)
        pltpu.async_copy(ef_hbm.at[pl.ds(2 * E + j * CH, CH)], idx_v.at[b, p, 1], sem)

    def _wait_idx(b, p):
        pltpu.make_async_copy(ef_hbm.at[pl.ds(0, CH)], idx_v.at[b, p, 0],
                              sem_i.at[b, p]).wait()
        pltpu.make_async_copy(ef_hbm.at[pl.ds(0, CH)], idx_v.at[b, p, 1],
                              sem_i.at[b, p]).wait()

    pltpu.sync_copy(
        xs_hbm.at[pl.ds(cc * NR + row0, RPT)], agg_sh.at[pl.ds(row0, RPT)]
    )
    plsc.subcore_barrier()

    def _wait_scatter(b, p):
        pltpu.make_async_copy(
            rows_v.at[b], agg_sh.at[idx_v.at[b, p, 1]], sem_s.at[b]
        ).wait()

    for b in range(NBUF):
        _fetch_idx(b * NS + sid, b, 0, sem_i.at[b, 0])

    def _run_group(g, phase, first=False):
        for b in range(NBUF):
            if not first:
                _wait_scatter(b, phase)
            _wait_idx(b, phase)
            pltpu.async_copy(
                xs_hbm.at[idx_v.at[b, phase, 0]], rows_v.at[b], sem_g.at[b]
            )
            jn = ((g + 1) * NBUF + b) * NS + sid

            @pl.when(jn < TPT * NS)
            def _():
                _fetch_idx(jn, b, 1 - phase, sem_i.at[b, 1 - phase])
        for b in range(NBUF):
            pltpu.make_async_copy(
                xs_hbm.at[idx_v.at[b, phase, 0]], rows_v.at[b], sem_g.at[b]
            ).wait()
            pltpu.async_copy(
                rows_v.at[b], agg_sh.at[idx_v.at[b, phase, 1]], sem_s.at[b],
                add=True,
            )

    @pl.loop(0, (NGRP - 1) // 2)
    def _group2(gg):
        for b in range(NBUF):
            @pl.when(gg > 0)
            def _():
                _wait_scatter(b, 0)
            _wait_idx(b, 0)
            pltpu.async_copy(
                xs_hbm.at[idx_v.at[b, 0, 0]], rows_v.at[b], sem_g.at[b]
            )
            jn = (gg * 2 * NBUF + NBUF + b) * NS + sid
            _fetch_idx(jn, b, 1, sem_i.at[b, 1])
        for b in range(NBUF):
            pltpu.make_async_copy(
                xs_hbm.at[idx_v.at[b, 0, 0]], rows_v.at[b], sem_g.at[b]
            ).wait()
            pltpu.async_copy(
                rows_v.at[b], agg_sh.at[idx_v.at[b, 0, 1]], sem_s.at[b],
                add=True,
            )
        _run_group(gg * 2 + 1, 1)

    _run_group(NGRP - 1, 0)

    for b in range(NBUF):
        _wait_scatter(b, 0)

    for r in range((NCHUNK - TPT * NS) // NS):
        j = (TPT + r) * NS + sid
        pltpu.sync_copy(ef_hbm.at[pl.ds(sbase + j * CH, CH)], idx_v.at[0, 0, 0])
        pltpu.sync_copy(ef_hbm.at[pl.ds(2 * E + j * CH, CH)], idx_v.at[0, 0, 1])
        pltpu.sync_copy(xs_hbm.at[idx_v.at[0, 0, 0]], rows_v.at[0])
        pltpu.sync_copy(rows_v.at[0], agg_sh.at[idx_v.at[0, 0, 1]], add=True)

    plsc.subcore_barrier()

    @pl.when(cc == 0)
    def _out0():
        pltpu.sync_copy(agg_sh.at[pl.ds(row0, RPT)], agg0_hbm.at[pl.ds(row0, RPT)])

    @pl.when(cc == 1)
    def _out1():
        pltpu.sync_copy(agg_sh.at[pl.ds(row0, RPT)], agg1_hbm.at[pl.ds(row0, RPT)])


def _tc_prep_body(x_ref, degp_ref, xs_ref, dinv_ref):
    deg = jnp.sum(degp_ref[...], axis=0) + 1.0
    dinv = lax.rsqrt(deg)
    dcol = dinv[:N].reshape(N, 1)
    dinv_ref[...] = dcol
    xs_ref[0:N, :] = x_ref[:, 0:FH] * dcol
    xs_ref[N:NR, :] = jnp.zeros((NR - N, FH), jnp.float32)
    xs_ref[NR : NR + N, :] = x_ref[:, FH:F] * dcol
    xs_ref[NR + N :, :] = jnp.zeros((NR - N, FH), jnp.float32)


_tc_prep = pl.pallas_call(
    _tc_prep_body,
    out_shape=[
        jax.ShapeDtypeStruct((NC * NR, FH), jnp.float32),
        jax.ShapeDtypeStruct((N, 1), jnp.float32),
    ],
)


def _tc_final_body(agg0_ref, agg1_ref, dinv_ref, w_ref, b_ref, g_ref, be_ref,
                   out_ref):
    dcol = dinv_ref[...]
    z0 = (agg0_ref[0:N, :] * dcol).astype(jnp.bfloat16)
    z1 = (agg1_ref[0:N, :] * dcol).astype(jnp.bfloat16)
    t = (
        jnp.dot(z0, w_ref[0:FH, :].astype(jnp.bfloat16),
                preferred_element_type=jnp.float32)
        + jnp.dot(z1, w_ref[FH:F, :].astype(jnp.bfloat16),
                  preferred_element_type=jnp.float32)
        + b_ref[...]
    )
    mean = jnp.mean(t, axis=0)
    var = jnp.mean(t * t, axis=0) - mean * mean
    o = (t - mean) * lax.rsqrt(var + EPS) * g_ref[...] + be_ref[...]
    out_ref[...] = jnp.where(o >= 0, o, ALPHA * o)


_tc_final = pl.pallas_call(
    _tc_final_body,
    out_shape=jax.ShapeDtypeStruct((N, F), jnp.float32),
)


def kernel(x, edge_idx, W, b, gamma, beta):
    ef2 = edge_idx.reshape(2 * E)
    ef3 = jnp.concatenate([edge_idx[0], edge_idx[0] + NR, edge_idx[1]])
    degp = _sc_degree(ef2).reshape(NC * NS, NP)
    xs, dinv = _tc_prep(x, degp)
    agg0, agg1 = _sc_aggregate(ef3, xs)
    return _tc_final(agg0, agg1, dinv, W, b, gamma, beta)


# speedup vs baseline: 1.0490x; 1.0128x over previous
import dataclasses
import functools

import jax
import jax.numpy as jnp
from jax import lax
from jax.experimental import pallas as pl
from jax.experimental.pallas import tpu as pltpu
from jax.experimental.pallas import tpu_sc as plsc

N = 10000
E = 160000
F = 256
FH = 128
NP = 10240
NR = 10240
ALPHA = 0.2
EPS = 1e-5
NC = 2
NS = 16
CH = 80
NCHUNK = E // CH
RPT = NR // NS

_sc_mesh = plsc.VectorSubcoreMesh(core_axis_name="c", subcore_axis_name="s")

_sc_cp = pltpu.CompilerParams()
if "needs_layout_passes" in pltpu.CompilerParams.__dataclass_fields__:
    _sc_cp = dataclasses.replace(_sc_cp, needs_layout_passes=False)


EPT = E // (NC * NS)
EPT_FULL = (EPT // 16) * 16


@functools.partial(
    pl.kernel,
    out_type=jax.ShapeDtypeStruct((NC * NS * NP,), jnp.float32),
    mesh=_sc_mesh,
    compiler_params=_sc_cp,
    scratch_types=[
        pltpu.VMEM((NP,), jnp.float32),
        pltpu.VMEM((EPT_FULL + 16,), jnp.int32),
    ],
)
def _sc_degree(ef_hbm, degp_hbm, hist_v, idx_v):
    cc = lax.axis_index("c")
    sid = lax.axis_index("s")
    ones = jnp.full((16,), 1.0, jnp.float32)

    @pl.loop(0, NP, step=128)
    def _zero(i):
        for u in range(8):
            hist_v[pl.ds(i + u * 16, 16)] = jnp.zeros((16,), jnp.float32)

    base_e = E + (cc * NS + sid) * EPT
    pltpu.sync_copy(ef_hbm.at[pl.ds(base_e, EPT)], idx_v.at[pl.ds(0, EPT)])

    @pl.loop(0, EPT_FULL, step=64)
    def _scat(i):
        for u in range(4):
            idx = idx_v[pl.ds(i + u * 16, 16)]
            plsc.addupdate_scatter(hist_v, [idx], ones)

    tail = EPT - EPT_FULL
    if tail:
        mask = jnp.arange(16, dtype=jnp.int32) < tail
        idx = jnp.where(mask, idx_v[pl.ds(EPT_FULL, 16)], 0)
        plsc.addupdate_scatter(hist_v, [idx], ones, mask=mask)

    pltpu.sync_copy(hist_v, degp_hbm.at[pl.ds((cc * NS + sid) * NP, NP)])


NBUF = 4
NGRP = 31
TPT = NGRP * NBUF


@functools.partial(
    pl.kernel,
    out_type=[
        jax.ShapeDtypeStruct((NR, FH), jnp.float32),
        jax.ShapeDtypeStruct((NR, FH), jnp.float32),
    ],
    mesh=_sc_mesh,
    compiler_params=_sc_cp,
    scratch_types=[
        pltpu.VMEM((NBUF, 2, 2, CH), jnp.int32),
        pltpu.VMEM((NBUF, CH, FH), jnp.float32),
        pltpu.VMEM_SHARED((NR, FH), jnp.float32),
        pltpu.SemaphoreType.DMA((NBUF, 2)),
        pltpu.SemaphoreType.DMA((NBUF,)),
        pltpu.SemaphoreType.DMA((NBUF,)),
    ],
)
def _sc_aggregate(ef_hbm, xs_hbm, agg0_hbm, agg1_hbm, idx_v, rows_v, agg_sh,
                  sem_i, sem_g, sem_s):
    cc = lax.axis_index("c")
    sid = lax.axis_index("s")
    row0 = sid * RPT
    sbase = cc * E

    def _fetch_idx(j, b, p, sem):
        pltpu.async_copy(ef_hbm.at[pl.ds(sbase + j * CH, CH)], idx_v.at[b, p, 0], sem)
        pltpu.async_copy(ef_hbm.at[pl.ds(2 * E + j * CH, CH)], idx_v.at[b, p, 1], sem)

    def _wait_idx(b, p):
        pltpu.make_async_copy(ef_hbm.at[pl.ds(0, CH)], idx_v.at[b, p, 0],
                              sem_i.at[b, p]).wait()
        pltpu.make_async_copy(ef_hbm.at[pl.ds(0, CH)], idx_v.at[b, p, 1],
                              sem_i.at[b, p]).wait()

    for b in range(NBUF):
        _fetch_idx(b * NS + sid, b, 0, sem_i.at[b, 0])
    pltpu.sync_copy(
        xs_hbm.at[pl.ds(cc * NR + row0, RPT)], agg_sh.at[pl.ds(row0, RPT)]
    )
    for b in range(NBUF):
        _wait_idx(b, 0)
        pltpu.async_copy(
            xs_hbm.at[idx_v.at[b, 0, 0]], rows_v.at[b], sem_g.at[b]
        )
        _fetch_idx((NBUF + b) * NS + sid, b, 1, sem_i.at[b, 1])
    plsc.subcore_barrier()

    def _wait_scatter(b, p):
        pltpu.make_async_copy(
            rows_v.at[b], agg_sh.at[idx_v.at[b, p, 1]], sem_s.at[b]
        ).wait()

    def _run_group(g, phase, first=False):
        for b in range(NBUF):
            if not first:
                _wait_scatter(b, phase)
            _wait_idx(b, phase)
            pltpu.async_copy(
                xs_hbm.at[idx_v.at[b, phase, 0]], rows_v.at[b], sem_g.at[b]
            )
            jn = ((g + 1) * NBUF + b) * NS + sid

            @pl.when(jn < TPT * NS)
            def _():
                _fetch_idx(jn, b, 1 - phase, sem_i.at[b, 1 - phase])
        for b in range(NBUF):
            pltpu.make_async_copy(
                xs_hbm.at[idx_v.at[b, phase, 0]], rows_v.at[b], sem_g.at[b]
            ).wait()
            pltpu.async_copy(
                rows_v.at[b], agg_sh.at[idx_v.at[b, phase, 1]], sem_s.at[b],
                add=True,
            )

    @pl.loop(0, (NGRP - 1) // 2)
    def _group2(gg):
        for b in range(NBUF):
            @pl.when(gg > 0)
            def _():
                _wait_scatter(b, 0)
                _wait_idx(b, 0)
                pltpu.async_copy(
                    xs_hbm.at[idx_v.at[b, 0, 0]], rows_v.at[b], sem_g.at[b]
                )
                _fetch_idx((gg * 2 * NBUF + NBUF + b) * NS + sid, b, 1,
                           sem_i.at[b, 1])
        for b in range(NBUF):
            pltpu.make_async_copy(
                xs_hbm.at[idx_v.at[b, 0, 0]], rows_v.at[b], sem_g.at[b]
            ).wait()
            pltpu.async_copy(
                rows_v.at[b], agg_sh.at[idx_v.at[b, 0, 1]], sem_s.at[b],
                add=True,
            )
        _run_group(gg * 2 + 1, 1)

    _run_group(NGRP - 1, 0)

    for b in range(NBUF):
        _wait_scatter(b, 0)

    for r in range((NCHUNK - TPT * NS) // NS):
        j = (TPT + r) * NS + sid
        pltpu.sync_copy(ef_hbm.at[pl.ds(sbase + j * CH, CH)], idx_v.at[0, 0, 0])
        pltpu.sync_copy(ef_hbm.at[pl.ds(2 * E + j * CH, CH)], idx_v.at[0, 0, 1])
        pltpu.sync_copy(xs_hbm.at[idx_v.at[0, 0, 0]], rows_v.at[0])
        pltpu.sync_copy(rows_v.at[0], agg_sh.at[idx_v.at[0, 0, 1]], add=True)

    plsc.subcore_barrier()

    @pl.when(cc == 0)
    def _out0():
        pltpu.sync_copy(agg_sh.at[pl.ds(row0, RPT)], agg0_hbm.at[pl.ds(row0, RPT)])

    @pl.when(cc == 1)
    def _out1():
        pltpu.sync_copy(agg_sh.at[pl.ds(row0, RPT)], agg1_hbm.at[pl.ds(row0, RPT)])


def _tc_prep_body(x_ref, degp_ref, xs_ref, dinv_ref):
    deg = jnp.sum(degp_ref[...], axis=0) + 1.0
    dinv = lax.rsqrt(deg)
    dcol = dinv[:N].reshape(N, 1)
    dinv_ref[...] = dcol
    xs_ref[0:N, :] = x_ref[:, 0:FH] * dcol
    xs_ref[N:NR, :] = jnp.zeros((NR - N, FH), jnp.float32)
    xs_ref[NR : NR + N, :] = x_ref[:, FH:F] * dcol
    xs_ref[NR + N :, :] = jnp.zeros((NR - N, FH), jnp.float32)


_tc_prep = pl.pallas_call(
    _tc_prep_body,
    out_shape=[
        jax.ShapeDtypeStruct((NC * NR, FH), jnp.float32),
        jax.ShapeDtypeStruct((N, 1), jnp.float32),
    ],
)


def _tc_final_body(agg0_ref, agg1_ref, dinv_ref, w_ref, b_ref, g_ref, be_ref,
                   out_ref):
    dcol = dinv_ref[...]
    z0 = (agg0_ref[0:N, :] * dcol).astype(jnp.bfloat16)
    z1 = (agg1_ref[0:N, :] * dcol).astype(jnp.bfloat16)
    t = (
        jnp.dot(z0, w_ref[0:FH, :].astype(jnp.bfloat16),
                preferred_element_type=jnp.float32)
        + jnp.dot(z1, w_ref[FH:F, :].astype(jnp.bfloat16),
                  preferred_element_type=jnp.float32)
        + b_ref[...]
    )
    mean = jnp.mean(t, axis=0)
    var = jnp.mean(t * t, axis=0) - mean * mean
    o = (t - mean) * lax.rsqrt(var + EPS) * g_ref[...] + be_ref[...]
    out_ref[...] = jnp.where(o >= 0, o, ALPHA * o)


_tc_final = pl.pallas_call(
    _tc_final_body,
    out_shape=jax.ShapeDtypeStruct((N, F), jnp.float32),
)


def kernel(x, edge_idx, W, b, gamma, beta):
    ef2 = edge_idx.reshape(2 * E)
    ef3 = jnp.concatenate([edge_idx[0], edge_idx[0] + NR, edge_idx[1]])
    degp = _sc_degree(ef2).reshape(NC * NS, NP)
    xs, dinv = _tc_prep(x, degp)
    agg0, agg1 = _sc_aggregate(ef3, xs)
    return _tc_final(agg0, agg1, dinv, W, b, gamma, beta)
